# baseline (speedup 1.0000x reference)
import functools

import jax
import jax.numpy as jnp
from jax import lax
from jax.experimental import pallas as pl
from jax.experimental.pallas import tpu as pltpu
from jax.experimental.pallas import tpu_sc as plsc

N = 10000
E = 320000
D = 128

NC = 2
NS = 16
K = 40
EPT = E // (NC * NS)
CH = EPT // K
NB = 5
NS_RING = 2 * NB
GROUPS = CH // NS_RING
GBYTES = K * D * 4
IBYTES = K * 4
RPT = 632
RPT_LAST = N - RPT * (NS - 1)
ZB = K
ZTAIL = RPT - (RPT // ZB) * ZB

MM_BLK = 2000


def _combine_mm_relu_kernel(p_ref, w_ref, o_ref):
    agg = p_ref[0] + p_ref[1]
    o_ref[...] = jnp.maximum(
        jnp.dot(agg, w_ref[...], preferred_element_type=jnp.float32), 0.0)


def _combine_mm_relu(partials, w):
    return pl.pallas_call(
        _combine_mm_relu_kernel,
        grid=(N // MM_BLK,),
        in_specs=[
            pl.BlockSpec((NC, MM_BLK, D), lambda i: (0, i, 0)),
            pl.BlockSpec((D, D), lambda i: (0, 0)),
        ],
        out_specs=pl.BlockSpec((MM_BLK, D), lambda i: (i, 0)),
        out_shape=jax.ShapeDtypeStruct((N, D), jnp.float32),
    )(partials, w)


_SC_MESH = plsc.VectorSubcoreMesh(
    core_axis_name="c", subcore_axis_name="s", num_cores=NC, num_subcores=NS)


@functools.partial(
    pl.kernel,
    out_type=jax.ShapeDtypeStruct((NC, N, D), jnp.float32),
    mesh=_SC_MESH,
    scratch_types=[
        pltpu.VMEM((NS_RING, K), jnp.int32),
        pltpu.VMEM((NB, K), jnp.int32),
        pltpu.VMEM((NB, K, D), jnp.float32),
        pltpu.VMEM_SHARED((N, D), jnp.float32),
        [pltpu.SemaphoreType.DMA] * NB,
        [pltpu.SemaphoreType.DMA] * NS_RING,
        [pltpu.SemaphoreType.DMA] * NB,
    ],
)
def _segsum_sc(h_hbm, src_hbm, dst_hbm, out_hbm,
               src_v, dst_v, rows_v, acc_sh, slot_sems, src_sems, ssems):
    cid = lax.axis_index("c")
    sid = lax.axis_index("s")
    ebase = (cid * NS + sid) * EPT
    row_base = pl.multiple_of(sid * RPT, 8)
    nrows = jnp.where(sid == NS - 1, RPT_LAST, RPT)

    z16 = jnp.zeros((16,), jnp.float32)

    @pl.loop(0, ZB * (D // 16))
    def _(i):
        rows_v[0, i // (D // 16), pl.ds((i % (D // 16)) * 16, 16)] = z16

    @pl.loop(0, nrows // ZB)
    def _(r):
        pltpu.async_copy(
            rows_v.at[0],
            acc_sh.at[pl.ds(pl.multiple_of(row_base + r * ZB, 8), ZB)],
            ssems[0])

    @pl.when(sid < NS - 1)
    def _():
        pltpu.async_copy(
            rows_v.at[0, pl.ds(0, ZTAIL)],
            acc_sh.at[pl.ds(pl.multiple_of(row_base + (RPT // ZB) * ZB, 8),
                            ZTAIL)],
            ssems[0])

    @pl.loop(0, nrows // ZB)
    def _(r):
        pltpu.make_async_copy(
            rows_v.at[0], acc_sh.at[pl.ds(row_base, ZB)], ssems[0]).wait()

    @pl.when(sid < NS - 1)
    def _():
        pltpu.make_async_copy(
            rows_v.at[0, pl.ds(0, ZTAIL)],
            acc_sh.at[pl.ds(row_base, ZTAIL)], ssems[0]).wait()

    plsc.subcore_barrier()

    def _src_fill(c, s):
        pltpu.async_copy(
            src_hbm.at[pl.ds(ebase + c * K, K)], src_v.at[s], src_sems[s])

    def _src_wait(s):
        pltpu.make_async_copy(
            src_hbm.at[pl.ds(0, K)], src_v.at[s], src_sems[s]).wait()

    def _scatter_wait(b):
        pltpu.make_async_copy(
            rows_v.at[b], acc_sh.at[dst_v.at[b]], ssems[b]).wait()

    for s in range(NS_RING):
        _src_fill(s, s)
    for c in range(NB):
        pltpu.async_copy(
            dst_hbm.at[pl.ds(ebase + c * K, K)], dst_v.at[c], slot_sems[c])
        _src_wait(c)
        pltpu.async_copy(h_hbm.at[src_v.at[c]], rows_v.at[c], slot_sems[c])

    @pl.loop(0, GROUPS)
    def _(g):
        for i in range(NS_RING):
            v = g * NS_RING + i
            b = i % NB
            bp = (i - 2) % NB
            sg = (i + NB - 2) % NS_RING
            sr = (i - 1) % NS_RING

            pltpu.make_async_copy(
                h_hbm.at[src_v.at[0]], rows_v.at[b], slot_sems[b]).wait()
            pltpu.make_async_copy(
                dst_hbm.at[pl.ds(0, K)], dst_v.at[b], slot_sems[b]).wait()

            if i <= 1:
                pl.when(g > 0)(lambda bp=bp: _scatter_wait(bp))
            else:
                _scatter_wait(bp)

            pltpu.async_copy(
                rows_v.at[b], acc_sh.at[dst_v.at[b]], ssems[b], add=True)

            def _refill(v=v, bp=bp, sg=sg):
                pltpu.async_copy(
                    dst_hbm.at[pl.ds(ebase + (v + NB - 2) * K, K)],
                    dst_v.at[bp], slot_sems[bp])
                _src_wait(sg)
                pltpu.async_copy(
                    h_hbm.at[src_v.at[sg]], rows_v.at[bp], slot_sems[bp])

            if i <= 1:
                pl.when(g > 0)(_refill)
            elif i <= NB + 1:
                _refill()
            else:
                pl.when(g < GROUPS - 1)(_refill)

            def _src_refill(v=v, sr=sr):
                _src_fill_dyn = pltpu.async_copy(
                    src_hbm.at[pl.ds(ebase + (v + NS_RING - 1) * K, K)],
                    src_v.at[sr], src_sems[sr])

            if i == 0:
                pl.when(g > 0)(_src_refill)
            else:
                pl.when(g < GROUPS - 1)(_src_refill)

    _scatter_wait((CH - 2) % NB)
    _scatter_wait((CH - 1) % NB)

    plsc.subcore_barrier()

    @pl.when(sid < NS - 1)
    def _():
        pltpu.sync_copy(acc_sh.at[pl.ds(row_base, RPT)],
                        out_hbm.at[cid, pl.ds(row_base, RPT)])

    @pl.when(sid == NS - 1)
    def _():
        pltpu.sync_copy(acc_sh.at[pl.ds(row_base, RPT_LAST)],
                        out_hbm.at[cid, pl.ds(row_base, RPT_LAST)])


def _segment_sum_partials(h, src, dst):
    return _segsum_sc(h, src, dst)


def kernel(x, edge_index, hyper_edge_index, W1, W2):
    src, dst = edge_index[0], edge_index[1]
    hsrc, hdst = hyper_edge_index[0], hyper_edge_index[1]

    p1 = _segment_sum_partials(x, src, dst)
    x1 = _combine_mm_relu(p1, W1)
    p2 = _segment_sum_partials(x1, hsrc, hdst)
    return _combine_mm_relu(p2, W2)


# speedup vs baseline: 1.1354x; 1.1354x over previous
import functools

import jax
import jax.numpy as jnp
from jax import lax
from jax.experimental import pallas as pl
from jax.experimental.pallas import tpu as pltpu
from jax.experimental.pallas import tpu_sc as plsc

N = 10000
E = 320000
D = 128

NC = 2
NS = 16
K = 40
EPT = E // (NC * NS)
CH = EPT // K
NB = 5
NS_RING = 2 * NB
GROUPS = CH // NS_RING
GBYTES = K * D * 4
IBYTES = K * 4
RPT = 632
RPT_LAST = N - RPT * (NS - 1)
ZB = K
ZTAIL = RPT - (RPT // ZB) * ZB

MM_BLK = 2000


def _combine_mm_relu_kernel(p_ref, w_ref, o_ref):
    agg = p_ref[0] + p_ref[1]
    o_ref[...] = jnp.maximum(
        jnp.dot(agg, w_ref[...], preferred_element_type=jnp.float32), 0.0)


def _combine_mm_relu(partials, w):
    return pl.pallas_call(
        _combine_mm_relu_kernel,
        grid=(N // MM_BLK,),
        in_specs=[
            pl.BlockSpec((NC, MM_BLK, D), lambda i: (0, i, 0)),
            pl.BlockSpec((D, D), lambda i: (0, 0)),
        ],
        out_specs=pl.BlockSpec((MM_BLK, D), lambda i: (i, 0)),
        out_shape=jax.ShapeDtypeStruct((N, D), jnp.float32),
    )(partials, w)


_SC_MESH = plsc.VectorSubcoreMesh(
    core_axis_name="c", subcore_axis_name="s", num_cores=NC, num_subcores=NS)


@functools.partial(
    pl.kernel,
    out_type=jax.ShapeDtypeStruct((NC, N, D), jnp.float32),
    mesh=_SC_MESH,
    scratch_types=[
        pltpu.VMEM((NS_RING, K), jnp.int32),
        pltpu.VMEM((NB, K), jnp.int32),
        pltpu.VMEM((NB, K, D), jnp.float32),
        pltpu.VMEM_SHARED((N, D), jnp.float32),
        [pltpu.SemaphoreType.DMA] * NB,
        [pltpu.SemaphoreType.DMA] * NS_RING,
        pltpu.SemaphoreType.DMA,
    ],
)
def _segsum_sc(h_hbm, src_hbm, dst_hbm, out_hbm,
               src_v, dst_v, rows_v, acc_sh, slot_sems, src_sems, ssem):
    cid = lax.axis_index("c")
    sid = lax.axis_index("s")
    ebase = (cid * NS + sid) * EPT
    row_base = pl.multiple_of(sid * RPT, 8)
    nrows = jnp.where(sid == NS - 1, RPT_LAST, RPT)

    z16 = jnp.zeros((16,), jnp.float32)

    @pl.loop(0, ZB * (D // 16))
    def _(i):
        rows_v[0, i // (D // 16), pl.ds((i % (D // 16)) * 16, 16)] = z16

    @pl.loop(0, nrows // ZB)
    def _(r):
        pltpu.async_copy(
            rows_v.at[0],
            acc_sh.at[pl.ds(pl.multiple_of(row_base + r * ZB, 8), ZB)],
            ssem)

    @pl.when(sid < NS - 1)
    def _():
        pltpu.async_copy(
            rows_v.at[0, pl.ds(0, ZTAIL)],
            acc_sh.at[pl.ds(pl.multiple_of(row_base + (RPT // ZB) * ZB, 8),
                            ZTAIL)],
            ssem)

    @pl.loop(0, nrows // ZB)
    def _(r):
        pltpu.make_async_copy(
            rows_v.at[0], acc_sh.at[pl.ds(row_base, ZB)], ssem).wait()

    @pl.when(sid < NS - 1)
    def _():
        pltpu.make_async_copy(
            rows_v.at[0, pl.ds(0, ZTAIL)],
            acc_sh.at[pl.ds(row_base, ZTAIL)], ssem).wait()

    plsc.subcore_barrier()

    def _src_fill(c, s):
        pltpu.async_copy(
            src_hbm.at[pl.ds(ebase + c * K, K)], src_v.at[s], src_sems[s])

    def _src_wait(s):
        pltpu.make_async_copy(
            src_hbm.at[pl.ds(0, K)], src_v.at[s], src_sems[s]).wait()

    def _scatter_wait(b):
        pltpu.make_async_copy(
            rows_v.at[b], acc_sh.at[dst_v.at[b]], ssem).wait()

    for s in range(NS_RING):
        _src_fill(s, s)
    for c in range(NB):
        pltpu.async_copy(
            dst_hbm.at[pl.ds(ebase + c * K, K)], dst_v.at[c], slot_sems[c])
        _src_wait(c)
        pltpu.async_copy(h_hbm.at[src_v.at[c]], rows_v.at[c], slot_sems[c])

    @pl.loop(0, GROUPS)
    def _(g):
        for i in range(NS_RING):
            v = g * NS_RING + i
            b = i % NB
            bp = (b - 1) % NB
            sg = (i + NB - 1) % NS_RING
            sr = (i - 1) % NS_RING

            pltpu.make_async_copy(
                h_hbm.at[src_v.at[0]], rows_v.at[b], slot_sems[b]).wait()
            pltpu.make_async_copy(
                dst_hbm.at[pl.ds(0, K)], dst_v.at[b], slot_sems[b]).wait()

            if i == 0:
                pl.when(g > 0)(lambda: _scatter_wait(NB - 1))
            else:
                _scatter_wait(bp)

            pltpu.async_copy(
                rows_v.at[b], acc_sh.at[dst_v.at[b]], ssem, add=True)

            def _refill(v=v, bp=bp, sg=sg):
                pltpu.async_copy(
                    dst_hbm.at[pl.ds(ebase + (v + NB - 1) * K, K)],
                    dst_v.at[bp], slot_sems[bp])
                _src_wait(sg)
                pltpu.async_copy(
                    h_hbm.at[src_v.at[sg]], rows_v.at[bp], slot_sems[bp])

            if i == 0:
                pl.when(g > 0)(_refill)
            elif i <= NB:
                _refill()
            else:
                pl.when(g < GROUPS - 1)(_refill)

            def _src_refill(v=v, sr=sr):
                _src_fill_dyn = pltpu.async_copy(
                    src_hbm.at[pl.ds(ebase + (v + NS_RING - 1) * K, K)],
                    src_v.at[sr], src_sems[sr])

            if i == 0:
                pl.when(g > 0)(_src_refill)
            else:
                pl.when(g < GROUPS - 1)(_src_refill)

    _scatter_wait((CH - 1) % NB)

    plsc.subcore_barrier()

    @pl.when(sid < NS - 1)
    def _():
        pltpu.sync_copy(acc_sh.at[pl.ds(row_base, RPT)],
                        out_hbm.at[cid, pl.ds(row_base, RPT)])

    @pl.when(sid == NS - 1)
    def _():
        pltpu.sync_copy(acc_sh.at[pl.ds(row_base, RPT_LAST)],
                        out_hbm.at[cid, pl.ds(row_base, RPT_LAST)])


def _segment_sum_partials(h, src, dst):
    return _segsum_sc(h, src, dst)


def kernel(x, edge_index, hyper_edge_index, W1, W2):
    src, dst = edge_index[0], edge_index[1]
    hsrc, hdst = hyper_edge_index[0], hyper_edge_index[1]

    p1 = _segment_sum_partials(x, src, dst)
    x1 = _combine_mm_relu(p1, W1)
    p2 = _segment_sum_partials(x1, hsrc, hdst)
    return _combine_mm_relu(p2, W2)
